# Initial kernel scaffold; baseline (speedup 1.0000x reference)
import jax
import jax.numpy as jnp
from jax.experimental import pallas as pl


def kernel(mean, color, qvec, svec, alpha, c2w):
    raise NotImplementedError("write your pallas kernel here")


# speedup vs baseline: 6.6631x; 6.6631x over previous
import functools

import jax
import jax.numpy as jnp
from jax import lax
from jax.experimental import pallas as pl
from jax.experimental.pallas import tpu as pltpu
from jax.experimental.pallas import tpu_sc as plsc

_S = 512
_NPIX = _S * _S
_NCORES = 2
_NSUB = 16
_NW = _NCORES * _NSUB
_PER_W = 16000
_NPAD = _NW * _PER_W
_ACC_N = _NPIX * 4
_CHUNK_G = 4000
_CROWS = _CHUNK_G * 4 // 128
_ZWORDS = _ACC_N // _NSUB

_BN = 2048


def _bf(v):
    return v.astype(jnp.bfloat16).astype(jnp.float32)


def _math_body(c2w_ref, x_ref, idx_ref, vals_ref):
    row = lambda i: x_ref[i:i + 1, :]
    mx, my, mz = row(0), row(1), row(2)
    cr, cg, cb = row(3), row(4), row(5)
    qw, qx, qy, qz = row(6), row(7), row(8), row(9)
    s0, s1, s2 = row(10), row(11), row(12)
    al = row(13)

    r = [[c2w_ref[i, j] for j in range(3)] for i in range(3)]
    rb = [[_bf(jnp.float32(r[i][j])) for j in range(3)] for i in range(3)]
    t0 = _bf(mx - c2w_ref[0, 3])
    t1 = _bf(my - c2w_ref[1, 3])
    t2 = _bf(mz - c2w_ref[2, 3])
    px = rb[0][0] * t0 + rb[1][0] * t1 + rb[2][0] * t2
    py = rb[0][1] * t0 + rb[1][1] * t1 + rb[2][1] * t2
    pz = rb[0][2] * t0 + rb[1][2] * t1 + rb[2][2] * t2
    z = jnp.maximum(pz, 1e-3)
    u = px / z
    v = py / z

    qn = jnp.sqrt(qw * qw + qx * qx + qy * qy + qz * qz) + 1e-8
    w_, x_, y_, z_ = qw / qn, qx / qn, qy / qn, qz / qn
    r00 = 1.0 - 2.0 * (y_ * y_ + z_ * z_)
    r01 = 2.0 * (x_ * y_ - w_ * z_)
    r02 = 2.0 * (x_ * z_ + w_ * y_)
    r10 = 2.0 * (x_ * y_ + w_ * z_)
    r11 = 1.0 - 2.0 * (x_ * x_ + z_ * z_)
    r12 = 2.0 * (y_ * z_ - w_ * x_)
    r20 = 2.0 * (x_ * z_ - w_ * y_)
    r21 = 2.0 * (y_ * z_ + w_ * x_)
    r22 = 1.0 - 2.0 * (x_ * x_ + y_ * y_)

    m00, m01, m02 = _bf(s0 * r00), _bf(s1 * r01), _bf(s2 * r02)
    m10, m11, m12 = _bf(s0 * r10), _bf(s1 * r11), _bf(s2 * r12)
    m20, m21, m22 = _bf(s0 * r20), _bf(s1 * r21), _bf(s2 * r22)
    s00 = m00 * m00 + m01 * m01 + m02 * m02
    s01 = m00 * m10 + m01 * m11 + m02 * m12
    s02 = m00 * m20 + m01 * m21 + m02 * m22
    s11 = m10 * m10 + m11 * m11 + m12 * m12
    s12 = m10 * m20 + m11 * m21 + m12 * m22
    s22 = m20 * m20 + m21 * m21 + m22 * m22

    inv_z = _bf(1.0 / z)
    j02 = _bf(-px / (z * z))
    j12 = _bf(-py / (z * z))
    a00 = inv_z * rb[0][0] + j02 * rb[0][2]
    a01 = inv_z * rb[1][0] + j02 * rb[1][2]
    a02 = inv_z * rb[2][0] + j02 * rb[2][2]
    a10 = inv_z * rb[0][1] + j12 * rb[0][2]
    a11 = inv_z * rb[1][1] + j12 * rb[1][2]
    a12 = inv_z * rb[2][1] + j12 * rb[2][2]

    ba00, ba01, ba02 = _bf(a00), _bf(a01), _bf(a02)
    ba10, ba11, ba12 = _bf(a10), _bf(a11), _bf(a12)
    bs00, bs01, bs02 = _bf(s00), _bf(s01), _bf(s02)
    bs11, bs12, bs22 = _bf(s11), _bf(s12), _bf(s22)
    t00 = ba00 * bs00 + ba01 * bs01 + ba02 * bs02
    t01 = ba00 * bs01 + ba01 * bs11 + ba02 * bs12
    t02 = ba00 * bs02 + ba01 * bs12 + ba02 * bs22
    t10 = ba10 * bs00 + ba11 * bs01 + ba12 * bs02
    t11 = ba10 * bs01 + ba11 * bs11 + ba12 * bs12
    t12 = ba10 * bs02 + ba11 * bs12 + ba12 * bs22
    bt00, bt01, bt02 = _bf(t00), _bf(t01), _bf(t02)
    bt10, bt11, bt12 = _bf(t10), _bf(t11), _bf(t12)
    c00 = bt00 * ba00 + bt01 * ba01 + bt02 * ba02
    c01 = bt00 * ba10 + bt01 * ba11 + bt02 * ba12
    c10 = bt10 * ba00 + bt11 * ba01 + bt12 * ba02
    c11 = bt10 * ba10 + bt11 * ba11 + bt12 * ba12
    det = c00 * c11 - c01 * c10
    wg = al / jnp.sqrt(1.0 + jnp.abs(det))

    fx = u * float(_S) + (_S / 2.0)
    fy = v * float(_S) + (_S / 2.0)
    ix = jnp.clip(jnp.floor(fx), 0.0, _S - 1).astype(jnp.int32)
    iy = jnp.clip(jnp.floor(fy), 0.0, _S - 1).astype(jnp.int32)
    e0 = (iy * _S + ix) * 4
    idx_ref[0:1, :] = e0
    idx_ref[1:2, :] = e0 + 1
    idx_ref[2:3, :] = e0 + 2
    idx_ref[3:4, :] = e0 + 3
    vals_ref[0:1, :] = wg * cr
    vals_ref[1:2, :] = wg * cg
    vals_ref[2:3, :] = wg * cb
    vals_ref[3:4, :] = wg


def _run_math(c2w, x16):
    grid = _NPAD // _BN
    return pl.pallas_call(
        _math_body,
        grid=(grid,),
        in_specs=[
            pl.BlockSpec(memory_space=pltpu.SMEM),
            pl.BlockSpec((16, _BN), lambda i: (0, i)),
        ],
        out_specs=[
            pl.BlockSpec((4, _BN), lambda i: (0, i)),
            pl.BlockSpec((4, _BN), lambda i: (0, i)),
        ],
        out_shape=[
            jax.ShapeDtypeStruct((4, _NPAD), jnp.int32),
            jax.ShapeDtypeStruct((4, _NPAD), jnp.float32),
        ],
    )(c2w, x16)


_SC_MESH = plsc.VectorSubcoreMesh(
    core_axis_name="c", subcore_axis_name="s",
    num_cores=_NCORES, num_subcores=_NSUB,
)


@functools.partial(
    pl.kernel,
    out_type=jax.ShapeDtypeStruct((_NCORES, _ACC_N), jnp.float32),
    mesh=_SC_MESH,
    scratch_types=[
        pltpu.VMEM((_CROWS, 128), jnp.int32),
        pltpu.VMEM((_CHUNK_G * 4,), jnp.float32),
        pltpu.VMEM_SHARED((_ACC_N,), jnp.float32),
    ],
    compiler_params=pltpu.CompilerParams(use_tc_tiling_on_sc=False),
)
def _sc_scatter(idx_hbm, vals_hbm, zeros_hbm, out_hbm, idx_v, vals_v, acc):
    c = lax.axis_index("c")
    s = lax.axis_index("s")
    wid = c * _NSUB + s
    zbase = s * _ZWORDS
    pltpu.sync_copy(zeros_hbm, acc.at[pl.ds(zbase, _ZWORDS)])
    plsc.subcore_barrier()

    for k in range(4):
        pltpu.sync_copy(idx_hbm.at[wid, k], idx_v)
        pltpu.sync_copy(vals_hbm.at[wid, k], vals_v)

        def body(j, carry):
            pltpu.sync_copy(
                vals_v.at[pl.ds(j * 128, 128)],
                acc.at[idx_v.at[j]],
                add=True,
            )
            return carry

        lax.fori_loop(0, _CROWS, body, 0)
    plsc.subcore_barrier()
    pltpu.sync_copy(acc.at[pl.ds(zbase, _ZWORDS)], out_hbm.at[c, pl.ds(zbase, _ZWORDS)])


_BP = 4096


def _final_body(p_ref, o_ref):
    a = p_ref[0] + p_ref[1]
    w = a[3:4, :]
    fac = (1.0 - jnp.exp(-w)) / (w + 1e-8)
    o_ref[...] = a[0:3, :] * fac


def _run_final(pt):
    grid = _NPIX // _BP
    return pl.pallas_call(
        _final_body,
        grid=(grid,),
        in_specs=[pl.BlockSpec((_NCORES, 4, _BP), lambda i: (0, 0, i))],
        out_specs=pl.BlockSpec((3, _BP), lambda i: (0, i)),
        out_shape=jax.ShapeDtypeStruct((3, _NPIX), jnp.float32),
    )(pt)


def kernel(mean, color, qvec, svec, alpha, c2w):
    n = mean.shape[0]
    x = jnp.concatenate(
        [mean, color, qvec, svec, alpha, jnp.zeros((n, 2), jnp.float32)], axis=1
    )
    x16 = jnp.pad(x.T, ((0, 0), (0, _NPAD - n)))
    idx4, vals4 = _run_math(c2w.astype(jnp.float32), x16)
    idx3 = idx4.T.reshape(_NW, 4, _CROWS, 128)
    vals = vals4.T.reshape(_NW, 4, _CHUNK_G * 4)
    zeros = jnp.zeros((_ZWORDS,), jnp.float32)
    part = _sc_scatter(idx3, vals, zeros)
    pt = jnp.transpose(part.reshape(_NCORES, _NPIX, 4), (0, 2, 1))
    rgb = _run_final(pt)
    return rgb.T.reshape(_S, _S, 3)


# speedup vs baseline: 22.9758x; 3.4482x over previous
import functools

import jax
import jax.numpy as jnp
from jax import lax
from jax.experimental import pallas as pl
from jax.experimental.pallas import tpu as pltpu
from jax.experimental.pallas import tpu_sc as plsc

_S = 512
_NPIX = _S * _S
_NCORES = 2
_NSUB = 16
_NW = _NCORES * _NSUB
_PER_W = 16000
_NPAD = _NW * _PER_W
_ACC_N = _NPIX * 4
_CHUNK_G = 4000
_CROWS = _CHUNK_G * 4 // 128
_ZWORDS = _ACC_N // _NSUB

_BN = 2048


def _bf(v):
    return v.astype(jnp.bfloat16).astype(jnp.float32)


def _math_body(c2w_ref, x_ref, idx_ref, vals_ref):
    row = lambda i: x_ref[i:i + 1, :]
    mx, my, mz = row(0), row(1), row(2)
    cr, cg, cb = row(3), row(4), row(5)
    qw, qx, qy, qz = row(6), row(7), row(8), row(9)
    s0, s1, s2 = row(10), row(11), row(12)
    al = row(13)

    r = [[c2w_ref[i, j] for j in range(3)] for i in range(3)]
    rb = [[_bf(jnp.float32(r[i][j])) for j in range(3)] for i in range(3)]
    t0 = _bf(mx - c2w_ref[0, 3])
    t1 = _bf(my - c2w_ref[1, 3])
    t2 = _bf(mz - c2w_ref[2, 3])
    px = rb[0][0] * t0 + rb[1][0] * t1 + rb[2][0] * t2
    py = rb[0][1] * t0 + rb[1][1] * t1 + rb[2][1] * t2
    pz = rb[0][2] * t0 + rb[1][2] * t1 + rb[2][2] * t2
    z = jnp.maximum(pz, 1e-3)
    u = px / z
    v = py / z

    qn = jnp.sqrt(qw * qw + qx * qx + qy * qy + qz * qz) + 1e-8
    w_, x_, y_, z_ = qw / qn, qx / qn, qy / qn, qz / qn
    r00 = 1.0 - 2.0 * (y_ * y_ + z_ * z_)
    r01 = 2.0 * (x_ * y_ - w_ * z_)
    r02 = 2.0 * (x_ * z_ + w_ * y_)
    r10 = 2.0 * (x_ * y_ + w_ * z_)
    r11 = 1.0 - 2.0 * (x_ * x_ + z_ * z_)
    r12 = 2.0 * (y_ * z_ - w_ * x_)
    r20 = 2.0 * (x_ * z_ - w_ * y_)
    r21 = 2.0 * (y_ * z_ + w_ * x_)
    r22 = 1.0 - 2.0 * (x_ * x_ + y_ * y_)

    m00, m01, m02 = _bf(s0 * r00), _bf(s1 * r01), _bf(s2 * r02)
    m10, m11, m12 = _bf(s0 * r10), _bf(s1 * r11), _bf(s2 * r12)
    m20, m21, m22 = _bf(s0 * r20), _bf(s1 * r21), _bf(s2 * r22)
    s00 = m00 * m00 + m01 * m01 + m02 * m02
    s01 = m00 * m10 + m01 * m11 + m02 * m12
    s02 = m00 * m20 + m01 * m21 + m02 * m22
    s11 = m10 * m10 + m11 * m11 + m12 * m12
    s12 = m10 * m20 + m11 * m21 + m12 * m22
    s22 = m20 * m20 + m21 * m21 + m22 * m22

    inv_z = _bf(1.0 / z)
    j02 = _bf(-px / (z * z))
    j12 = _bf(-py / (z * z))
    a00 = inv_z * rb[0][0] + j02 * rb[0][2]
    a01 = inv_z * rb[1][0] + j02 * rb[1][2]
    a02 = inv_z * rb[2][0] + j02 * rb[2][2]
    a10 = inv_z * rb[0][1] + j12 * rb[0][2]
    a11 = inv_z * rb[1][1] + j12 * rb[1][2]
    a12 = inv_z * rb[2][1] + j12 * rb[2][2]

    ba00, ba01, ba02 = _bf(a00), _bf(a01), _bf(a02)
    ba10, ba11, ba12 = _bf(a10), _bf(a11), _bf(a12)
    bs00, bs01, bs02 = _bf(s00), _bf(s01), _bf(s02)
    bs11, bs12, bs22 = _bf(s11), _bf(s12), _bf(s22)
    t00 = ba00 * bs00 + ba01 * bs01 + ba02 * bs02
    t01 = ba00 * bs01 + ba01 * bs11 + ba02 * bs12
    t02 = ba00 * bs02 + ba01 * bs12 + ba02 * bs22
    t10 = ba10 * bs00 + ba11 * bs01 + ba12 * bs02
    t11 = ba10 * bs01 + ba11 * bs11 + ba12 * bs12
    t12 = ba10 * bs02 + ba11 * bs12 + ba12 * bs22
    bt00, bt01, bt02 = _bf(t00), _bf(t01), _bf(t02)
    bt10, bt11, bt12 = _bf(t10), _bf(t11), _bf(t12)
    c00 = bt00 * ba00 + bt01 * ba01 + bt02 * ba02
    c01 = bt00 * ba10 + bt01 * ba11 + bt02 * ba12
    c10 = bt10 * ba00 + bt11 * ba01 + bt12 * ba02
    c11 = bt10 * ba10 + bt11 * ba11 + bt12 * ba12
    det = c00 * c11 - c01 * c10
    wg = al / jnp.sqrt(1.0 + jnp.abs(det))

    fx = u * float(_S) + (_S / 2.0)
    fy = v * float(_S) + (_S / 2.0)
    ix = jnp.clip(jnp.floor(fx), 0.0, _S - 1).astype(jnp.int32)
    iy = jnp.clip(jnp.floor(fy), 0.0, _S - 1).astype(jnp.int32)
    flat = iy * _S + ix
    idx_ref[0:1, :] = flat
    idx_ref[1:2, :] = flat + _NPIX
    idx_ref[2:3, :] = flat + 2 * _NPIX
    idx_ref[3:4, :] = flat + 3 * _NPIX
    vals_ref[0:1, :] = wg * cr
    vals_ref[1:2, :] = wg * cg
    vals_ref[2:3, :] = wg * cb
    vals_ref[3:4, :] = wg


def _run_math(c2w, x16):
    grid = _NPAD // _BN
    return pl.pallas_call(
        _math_body,
        grid=(grid,),
        in_specs=[
            pl.BlockSpec(memory_space=pltpu.SMEM),
            pl.BlockSpec((16, _BN), lambda i: (0, i)),
        ],
        out_specs=[
            pl.BlockSpec((4, _BN), lambda i: (0, i)),
            pl.BlockSpec((4, _BN), lambda i: (0, i)),
        ],
        out_shape=[
            jax.ShapeDtypeStruct((4, _NPAD), jnp.int32),
            jax.ShapeDtypeStruct((4, _NPAD), jnp.float32),
        ],
    )(c2w, x16)


_SC_MESH = plsc.VectorSubcoreMesh(
    core_axis_name="c", subcore_axis_name="s",
    num_cores=_NCORES, num_subcores=_NSUB,
)


@functools.partial(
    pl.kernel,
    out_type=jax.ShapeDtypeStruct((_NCORES, _ACC_N), jnp.float32),
    mesh=_SC_MESH,
    scratch_types=[
        pltpu.VMEM((_CROWS, 128), jnp.int32),
        pltpu.VMEM((_CHUNK_G * 4,), jnp.float32),
        pltpu.VMEM_SHARED((_ACC_N,), jnp.float32),
    ],
    compiler_params=pltpu.CompilerParams(use_tc_tiling_on_sc=False),
)
def _sc_scatter(idx_hbm, vals_hbm, zeros_hbm, out_hbm, idx_v, vals_v, acc):
    c = lax.axis_index("c")
    s = lax.axis_index("s")
    wid = c * _NSUB + s
    zbase = s * _ZWORDS
    pltpu.sync_copy(zeros_hbm, acc.at[pl.ds(zbase, _ZWORDS)])
    plsc.subcore_barrier()

    for k in range(4):
        pltpu.sync_copy(idx_hbm.at[wid, k], idx_v)
        pltpu.sync_copy(vals_hbm.at[wid, k], vals_v)

        def body(j, carry):
            pltpu.sync_copy(
                vals_v.at[pl.ds(j * 128, 128)],
                acc.at[idx_v.at[j]],
                add=True,
            )
            return carry

        lax.fori_loop(0, _CROWS, body, 0)
    plsc.subcore_barrier()
    pltpu.sync_copy(acc.at[pl.ds(zbase, _ZWORDS)], out_hbm.at[c, pl.ds(zbase, _ZWORDS)])


_BP = 4096


def _final_body(p_ref, o_ref):
    a = p_ref[0] + p_ref[1]
    w = a[3:4, :]
    fac = (1.0 - jnp.exp(-w)) / (w + 1e-8)
    o_ref[...] = a[0:3, :] * fac


def _run_final(pt):
    grid = _NPIX // _BP
    return pl.pallas_call(
        _final_body,
        grid=(grid,),
        in_specs=[pl.BlockSpec((_NCORES, 4, _BP), lambda i: (0, 0, i))],
        out_specs=pl.BlockSpec((3, _BP), lambda i: (0, i)),
        out_shape=jax.ShapeDtypeStruct((3, _NPIX), jnp.float32),
    )(pt)


def kernel(mean, color, qvec, svec, alpha, c2w):
    n = mean.shape[0]
    x = jnp.concatenate(
        [mean, color, qvec, svec, alpha, jnp.zeros((n, 2), jnp.float32)], axis=1
    )
    x16 = jnp.pad(x.T, ((0, 0), (0, _NPAD - n)))
    idx4, vals4 = _run_math(c2w.astype(jnp.float32), x16)
    idx3 = idx4.reshape(_NW, 4, _CROWS, 128)
    vals = vals4.reshape(_NW, 4, _CHUNK_G * 4)
    zeros = jnp.zeros((_ZWORDS,), jnp.float32)
    part = _sc_scatter(idx3, vals, zeros)
    rgb = _run_final(part.reshape(_NCORES, 4, _NPIX))
    return rgb.T.reshape(_S, _S, 3)


# speedup vs baseline: 28.2426x; 1.2292x over previous
import functools

import jax
import jax.numpy as jnp
from jax import lax
from jax.experimental import pallas as pl
from jax.experimental.pallas import tpu as pltpu
from jax.experimental.pallas import tpu_sc as plsc

_S = 512
_NPIX = _S * _S
_NCORES = 2
_NSUB = 16
_NW = _NCORES * _NSUB
_PER_W = 16000
_NPAD = _NW * _PER_W
_ACC_N = _NPIX * 4
_CHUNK_G = 4000
_CROWS = _CHUNK_G * 4 // 128
_ZWORDS = _ACC_N // _NSUB

_BN = 4096


def _bf(v):
    return v.astype(jnp.bfloat16).astype(jnp.float32)


def _math_body(c2w_ref, x_ref, idx_ref, vals_ref):
    row = lambda i: x_ref[i]
    mx, my, mz = row(0), row(1), row(2)
    cr, cg, cb = row(3), row(4), row(5)
    qw, qx, qy, qz = row(6), row(7), row(8), row(9)
    s0, s1, s2 = row(10), row(11), row(12)
    al = row(13)

    r = [[c2w_ref[i, j] for j in range(3)] for i in range(3)]
    rb = [[_bf(jnp.float32(r[i][j])) for j in range(3)] for i in range(3)]
    t0 = _bf(mx - c2w_ref[0, 3])
    t1 = _bf(my - c2w_ref[1, 3])
    t2 = _bf(mz - c2w_ref[2, 3])
    px = rb[0][0] * t0 + rb[1][0] * t1 + rb[2][0] * t2
    py = rb[0][1] * t0 + rb[1][1] * t1 + rb[2][1] * t2
    pz = rb[0][2] * t0 + rb[1][2] * t1 + rb[2][2] * t2
    z = jnp.maximum(pz, 1e-3)
    u = px / z
    v = py / z

    qn = jnp.sqrt(qw * qw + qx * qx + qy * qy + qz * qz) + 1e-8
    w_, x_, y_, z_ = qw / qn, qx / qn, qy / qn, qz / qn
    r00 = 1.0 - 2.0 * (y_ * y_ + z_ * z_)
    r01 = 2.0 * (x_ * y_ - w_ * z_)
    r02 = 2.0 * (x_ * z_ + w_ * y_)
    r10 = 2.0 * (x_ * y_ + w_ * z_)
    r11 = 1.0 - 2.0 * (x_ * x_ + z_ * z_)
    r12 = 2.0 * (y_ * z_ - w_ * x_)
    r20 = 2.0 * (x_ * z_ - w_ * y_)
    r21 = 2.0 * (y_ * z_ + w_ * x_)
    r22 = 1.0 - 2.0 * (x_ * x_ + y_ * y_)

    m00, m01, m02 = _bf(s0 * r00), _bf(s1 * r01), _bf(s2 * r02)
    m10, m11, m12 = _bf(s0 * r10), _bf(s1 * r11), _bf(s2 * r12)
    m20, m21, m22 = _bf(s0 * r20), _bf(s1 * r21), _bf(s2 * r22)
    s00 = m00 * m00 + m01 * m01 + m02 * m02
    s01 = m00 * m10 + m01 * m11 + m02 * m12
    s02 = m00 * m20 + m01 * m21 + m02 * m22
    s11 = m10 * m10 + m11 * m11 + m12 * m12
    s12 = m10 * m20 + m11 * m21 + m12 * m22
    s22 = m20 * m20 + m21 * m21 + m22 * m22

    inv_z = _bf(1.0 / z)
    j02 = _bf(-px / (z * z))
    j12 = _bf(-py / (z * z))
    a00 = inv_z * rb[0][0] + j02 * rb[0][2]
    a01 = inv_z * rb[1][0] + j02 * rb[1][2]
    a02 = inv_z * rb[2][0] + j02 * rb[2][2]
    a10 = inv_z * rb[0][1] + j12 * rb[0][2]
    a11 = inv_z * rb[1][1] + j12 * rb[1][2]
    a12 = inv_z * rb[2][1] + j12 * rb[2][2]

    ba00, ba01, ba02 = _bf(a00), _bf(a01), _bf(a02)
    ba10, ba11, ba12 = _bf(a10), _bf(a11), _bf(a12)
    bs00, bs01, bs02 = _bf(s00), _bf(s01), _bf(s02)
    bs11, bs12, bs22 = _bf(s11), _bf(s12), _bf(s22)
    t00 = ba00 * bs00 + ba01 * bs01 + ba02 * bs02
    t01 = ba00 * bs01 + ba01 * bs11 + ba02 * bs12
    t02 = ba00 * bs02 + ba01 * bs12 + ba02 * bs22
    t10 = ba10 * bs00 + ba11 * bs01 + ba12 * bs02
    t11 = ba10 * bs01 + ba11 * bs11 + ba12 * bs12
    t12 = ba10 * bs02 + ba11 * bs12 + ba12 * bs22
    bt00, bt01, bt02 = _bf(t00), _bf(t01), _bf(t02)
    bt10, bt11, bt12 = _bf(t10), _bf(t11), _bf(t12)
    c00 = bt00 * ba00 + bt01 * ba01 + bt02 * ba02
    c01 = bt00 * ba10 + bt01 * ba11 + bt02 * ba12
    c10 = bt10 * ba00 + bt11 * ba01 + bt12 * ba02
    c11 = bt10 * ba10 + bt11 * ba11 + bt12 * ba12
    det = c00 * c11 - c01 * c10
    wg = al / jnp.sqrt(1.0 + jnp.abs(det))

    fx = u * float(_S) + (_S / 2.0)
    fy = v * float(_S) + (_S / 2.0)
    ix = jnp.clip(jnp.floor(fx), 0.0, _S - 1).astype(jnp.int32)
    iy = jnp.clip(jnp.floor(fy), 0.0, _S - 1).astype(jnp.int32)
    flat = iy * _S + ix
    idx_ref[0] = flat
    idx_ref[1] = flat + _NPIX
    idx_ref[2] = flat + 2 * _NPIX
    idx_ref[3] = flat + 3 * _NPIX
    vals_ref[0] = wg * cr
    vals_ref[1] = wg * cg
    vals_ref[2] = wg * cb
    vals_ref[3] = wg


def _run_math(c2w, x16):
    nb = _NPAD // 1024
    b1 = _BN // 1024
    grid = nb // b1
    idx4, vals4 = pl.pallas_call(
        _math_body,
        grid=(grid,),
        in_specs=[
            pl.BlockSpec(memory_space=pltpu.SMEM),
            pl.BlockSpec((16, b1, 8, 128), lambda i: (0, i, 0, 0)),
        ],
        out_specs=[
            pl.BlockSpec((4, b1, 8, 128), lambda i: (0, i, 0, 0)),
            pl.BlockSpec((4, b1, 8, 128), lambda i: (0, i, 0, 0)),
        ],
        out_shape=[
            jax.ShapeDtypeStruct((4, nb, 8, 128), jnp.int32),
            jax.ShapeDtypeStruct((4, nb, 8, 128), jnp.float32),
        ],
    )(c2w, x16.reshape(16, nb, 8, 128))
    return idx4.reshape(4, _NPAD), vals4.reshape(4, _NPAD)


_SC_MESH = plsc.VectorSubcoreMesh(
    core_axis_name="c", subcore_axis_name="s",
    num_cores=_NCORES, num_subcores=_NSUB,
)


@functools.partial(
    pl.kernel,
    out_type=jax.ShapeDtypeStruct((_NCORES, _ACC_N), jnp.float32),
    mesh=_SC_MESH,
    scratch_types=[
        pltpu.VMEM((_CROWS, 128), jnp.int32),
        pltpu.VMEM((_CHUNK_G * 4,), jnp.float32),
        pltpu.VMEM_SHARED((_ACC_N,), jnp.float32),
    ],
    compiler_params=pltpu.CompilerParams(use_tc_tiling_on_sc=False),
)
def _sc_scatter(idx_hbm, vals_hbm, zeros_hbm, out_hbm, idx_v, vals_v, acc):
    c = lax.axis_index("c")
    s = lax.axis_index("s")
    wid = c * _NSUB + s
    zbase = s * _ZWORDS
    pltpu.sync_copy(zeros_hbm, acc.at[pl.ds(zbase, _ZWORDS)])
    plsc.subcore_barrier()

    for k in range(4):
        pltpu.sync_copy(idx_hbm.at[wid, k], idx_v)
        pltpu.sync_copy(vals_hbm.at[wid, k], vals_v)

        def body(j, carry):
            pltpu.sync_copy(
                vals_v.at[pl.ds(j * 128, 128)],
                acc.at[idx_v.at[j]],
                add=True,
            )
            return carry

        lax.fori_loop(0, _CROWS, body, 0)
    plsc.subcore_barrier()
    pltpu.sync_copy(acc.at[pl.ds(zbase, _ZWORDS)], out_hbm.at[c, pl.ds(zbase, _ZWORDS)])


_BP = 4096


def _final_body(p_ref, o_ref):
    w = p_ref[0, 3] + p_ref[1, 3]
    fac = (1.0 - jnp.exp(-w)) / (w + 1e-8)
    o_ref[0] = (p_ref[0, 0] + p_ref[1, 0]) * fac
    o_ref[1] = (p_ref[0, 1] + p_ref[1, 1]) * fac
    o_ref[2] = (p_ref[0, 2] + p_ref[1, 2]) * fac


def _run_final(pt):
    nb = _NPIX // 1024
    b3 = _BP // 1024
    grid = nb // b3
    rgb = pl.pallas_call(
        _final_body,
        grid=(grid,),
        in_specs=[pl.BlockSpec((_NCORES, 4, b3, 8, 128), lambda i: (0, 0, i, 0, 0))],
        out_specs=pl.BlockSpec((3, b3, 8, 128), lambda i: (0, i, 0, 0)),
        out_shape=jax.ShapeDtypeStruct((3, nb, 8, 128), jnp.float32),
    )(pt)
    return rgb.reshape(3, _NPIX)


def kernel(mean, color, qvec, svec, alpha, c2w):
    n = mean.shape[0]
    x = jnp.concatenate(
        [mean, color, qvec, svec, alpha, jnp.zeros((n, 2), jnp.float32)], axis=1
    )
    x16 = jnp.pad(x.T, ((0, 0), (0, _NPAD - n)))
    idx4, vals4 = _run_math(c2w.astype(jnp.float32), x16)
    idx3 = idx4.reshape(_NW, 4, _CROWS, 128)
    vals = vals4.reshape(_NW, 4, _CHUNK_G * 4)
    zeros = jnp.zeros((_ZWORDS,), jnp.float32)
    part = _sc_scatter(idx3, vals, zeros)
    rgb = _run_final(part.reshape(_NCORES, 4, _NPIX // 1024, 8, 128))
    return rgb.T.reshape(_S, _S, 3)


# speedup vs baseline: 39.5399x; 1.4000x over previous
import functools

import jax
import jax.numpy as jnp
from jax import lax
from jax.experimental import pallas as pl
from jax.experimental.pallas import tpu as pltpu
from jax.experimental.pallas import tpu_sc as plsc

_S = 512
_NPIX = _S * _S
_NCORES = 2
_NSUB = 16
_NW = _NCORES * _NSUB
_PER_W = 16000
_NPAD = _NW * _PER_W
_ACC_N = _NPIX * 4
_CHUNK_G = 4000
_CROWS = _CHUNK_G * 4 // 128
_ZWORDS = _ACC_N // _NSUB

_BN = 4096


def _bf(v):
    return v.astype(jnp.bfloat16).astype(jnp.float32)


def _math_body(c2w_ref, m_ref, col_ref, q_ref, sv_ref, a_ref, idx_ref, vals_ref):
    mx, my, mz = m_ref[:, 0, :], m_ref[:, 1, :], m_ref[:, 2, :]
    cr, cg, cb = col_ref[:, 0, :], col_ref[:, 1, :], col_ref[:, 2, :]
    qw, qx, qy, qz = q_ref[:, 0, :], q_ref[:, 1, :], q_ref[:, 2, :], q_ref[:, 3, :]
    s0, s1, s2 = sv_ref[:, 0, :], sv_ref[:, 1, :], sv_ref[:, 2, :]
    al = a_ref[:, 0, :]

    r = [[c2w_ref[i, j] for j in range(3)] for i in range(3)]
    rb = [[_bf(jnp.float32(r[i][j])) for j in range(3)] for i in range(3)]
    t0 = _bf(mx - c2w_ref[0, 3])
    t1 = _bf(my - c2w_ref[1, 3])
    t2 = _bf(mz - c2w_ref[2, 3])
    px = rb[0][0] * t0 + rb[1][0] * t1 + rb[2][0] * t2
    py = rb[0][1] * t0 + rb[1][1] * t1 + rb[2][1] * t2
    pz = rb[0][2] * t0 + rb[1][2] * t1 + rb[2][2] * t2
    z = jnp.maximum(pz, 1e-3)
    u = px / z
    v = py / z

    qn = jnp.sqrt(qw * qw + qx * qx + qy * qy + qz * qz) + 1e-8
    w_, x_, y_, z_ = qw / qn, qx / qn, qy / qn, qz / qn
    r00 = 1.0 - 2.0 * (y_ * y_ + z_ * z_)
    r01 = 2.0 * (x_ * y_ - w_ * z_)
    r02 = 2.0 * (x_ * z_ + w_ * y_)
    r10 = 2.0 * (x_ * y_ + w_ * z_)
    r11 = 1.0 - 2.0 * (x_ * x_ + z_ * z_)
    r12 = 2.0 * (y_ * z_ - w_ * x_)
    r20 = 2.0 * (x_ * z_ - w_ * y_)
    r21 = 2.0 * (y_ * z_ + w_ * x_)
    r22 = 1.0 - 2.0 * (x_ * x_ + y_ * y_)

    m00, m01, m02 = _bf(s0 * r00), _bf(s1 * r01), _bf(s2 * r02)
    m10, m11, m12 = _bf(s0 * r10), _bf(s1 * r11), _bf(s2 * r12)
    m20, m21, m22 = _bf(s0 * r20), _bf(s1 * r21), _bf(s2 * r22)
    s00 = m00 * m00 + m01 * m01 + m02 * m02
    s01 = m00 * m10 + m01 * m11 + m02 * m12
    s02 = m00 * m20 + m01 * m21 + m02 * m22
    s11 = m10 * m10 + m11 * m11 + m12 * m12
    s12 = m10 * m20 + m11 * m21 + m12 * m22
    s22 = m20 * m20 + m21 * m21 + m22 * m22

    inv_z = _bf(1.0 / z)
    j02 = _bf(-px / (z * z))
    j12 = _bf(-py / (z * z))
    a00 = inv_z * rb[0][0] + j02 * rb[0][2]
    a01 = inv_z * rb[1][0] + j02 * rb[1][2]
    a02 = inv_z * rb[2][0] + j02 * rb[2][2]
    a10 = inv_z * rb[0][1] + j12 * rb[0][2]
    a11 = inv_z * rb[1][1] + j12 * rb[1][2]
    a12 = inv_z * rb[2][1] + j12 * rb[2][2]

    ba00, ba01, ba02 = _bf(a00), _bf(a01), _bf(a02)
    ba10, ba11, ba12 = _bf(a10), _bf(a11), _bf(a12)
    bs00, bs01, bs02 = _bf(s00), _bf(s01), _bf(s02)
    bs11, bs12, bs22 = _bf(s11), _bf(s12), _bf(s22)
    t00 = ba00 * bs00 + ba01 * bs01 + ba02 * bs02
    t01 = ba00 * bs01 + ba01 * bs11 + ba02 * bs12
    t02 = ba00 * bs02 + ba01 * bs12 + ba02 * bs22
    t10 = ba10 * bs00 + ba11 * bs01 + ba12 * bs02
    t11 = ba10 * bs01 + ba11 * bs11 + ba12 * bs12
    t12 = ba10 * bs02 + ba11 * bs12 + ba12 * bs22
    bt00, bt01, bt02 = _bf(t00), _bf(t01), _bf(t02)
    bt10, bt11, bt12 = _bf(t10), _bf(t11), _bf(t12)
    c00 = bt00 * ba00 + bt01 * ba01 + bt02 * ba02
    c01 = bt00 * ba10 + bt01 * ba11 + bt02 * ba12
    c10 = bt10 * ba00 + bt11 * ba01 + bt12 * ba02
    c11 = bt10 * ba10 + bt11 * ba11 + bt12 * ba12
    det = c00 * c11 - c01 * c10
    wg = al / jnp.sqrt(1.0 + jnp.abs(det))

    fx = u * float(_S) + (_S / 2.0)
    fy = v * float(_S) + (_S / 2.0)
    ix = jnp.clip(jnp.floor(fx), 0.0, _S - 1).astype(jnp.int32)
    iy = jnp.clip(jnp.floor(fy), 0.0, _S - 1).astype(jnp.int32)
    flat = iy * _S + ix
    idx_ref[0] = flat
    idx_ref[1] = flat + _NPIX
    idx_ref[2] = flat + 2 * _NPIX
    idx_ref[3] = flat + 3 * _NPIX
    vals_ref[0] = wg * cr
    vals_ref[1] = wg * cg
    vals_ref[2] = wg * cb
    vals_ref[3] = wg


def _run_math(c2w, mean_t, color_t, qvec_t, svec_t, alpha_t):
    ng = _NPAD // 128
    bg = _BN // 128
    grid = ng // bg
    inspec = lambda k: pl.BlockSpec((bg, k, 128), lambda i: (i, 0, 0))
    idx4, vals4 = pl.pallas_call(
        _math_body,
        grid=(grid,),
        in_specs=[
            pl.BlockSpec(memory_space=pltpu.SMEM),
            inspec(3), inspec(3), inspec(4), inspec(3), inspec(1),
        ],
        out_specs=[
            pl.BlockSpec((4, bg, 128), lambda i: (0, i, 0)),
            pl.BlockSpec((4, bg, 128), lambda i: (0, i, 0)),
        ],
        out_shape=[
            jax.ShapeDtypeStruct((4, ng, 128), jnp.int32),
            jax.ShapeDtypeStruct((4, ng, 128), jnp.float32),
        ],
    )(c2w, mean_t, color_t, qvec_t, svec_t, alpha_t)
    return idx4.reshape(4, _NPAD), vals4.reshape(4, _NPAD)


_SC_MESH = plsc.VectorSubcoreMesh(
    core_axis_name="c", subcore_axis_name="s",
    num_cores=_NCORES, num_subcores=_NSUB,
)


@functools.partial(
    pl.kernel,
    out_type=jax.ShapeDtypeStruct((_NCORES, _ACC_N), jnp.float32),
    mesh=_SC_MESH,
    scratch_types=[
        pltpu.VMEM((_CROWS, 128), jnp.int32),
        pltpu.VMEM((_CHUNK_G * 4,), jnp.float32),
        pltpu.VMEM_SHARED((_ACC_N,), jnp.float32),
    ],
    compiler_params=pltpu.CompilerParams(use_tc_tiling_on_sc=False),
)
def _sc_scatter(idx_hbm, vals_hbm, zeros_hbm, out_hbm, idx_v, vals_v, acc):
    c = lax.axis_index("c")
    s = lax.axis_index("s")
    wid = c * _NSUB + s
    zbase = s * _ZWORDS
    pltpu.sync_copy(zeros_hbm, acc.at[pl.ds(zbase, _ZWORDS)])
    plsc.subcore_barrier()

    for k in range(4):
        pltpu.sync_copy(idx_hbm.at[wid, k], idx_v)
        pltpu.sync_copy(vals_hbm.at[wid, k], vals_v)

        def body(j, carry):
            pltpu.sync_copy(
                vals_v.at[pl.ds(j * 128, 128)],
                acc.at[idx_v.at[j]],
                add=True,
            )
            return carry

        lax.fori_loop(0, _CROWS, body, 0)
    plsc.subcore_barrier()
    pltpu.sync_copy(acc.at[pl.ds(zbase, _ZWORDS)], out_hbm.at[c, pl.ds(zbase, _ZWORDS)])


_BP = 4096


def _final_body(p_ref, o_ref):
    w = p_ref[0, 3] + p_ref[1, 3]
    fac = (1.0 - jnp.exp(-w)) / (w + 1e-8)
    o_ref[0] = (p_ref[0, 0] + p_ref[1, 0]) * fac
    o_ref[1] = (p_ref[0, 1] + p_ref[1, 1]) * fac
    o_ref[2] = (p_ref[0, 2] + p_ref[1, 2]) * fac


def _run_final(pt):
    nb = _NPIX // 1024
    b3 = _BP // 1024
    grid = nb // b3
    rgb = pl.pallas_call(
        _final_body,
        grid=(grid,),
        in_specs=[pl.BlockSpec((_NCORES, 4, b3, 8, 128), lambda i: (0, 0, i, 0, 0))],
        out_specs=pl.BlockSpec((3, b3, 8, 128), lambda i: (0, i, 0, 0)),
        out_shape=jax.ShapeDtypeStruct((3, nb, 8, 128), jnp.float32),
    )(pt)
    return rgb.reshape(3, _NPIX)


def _tilt(a, n):
    a = jnp.pad(a, ((0, _NPAD - n), (0, 0)))
    return a.reshape(_NPAD // 128, 128, a.shape[1]).transpose(0, 2, 1)


def kernel(mean, color, qvec, svec, alpha, c2w):
    n = mean.shape[0]
    idx4, vals4 = _run_math(
        c2w.astype(jnp.float32),
        _tilt(mean, n), _tilt(color, n), _tilt(qvec, n),
        _tilt(svec, n), _tilt(alpha, n),
    )
    idx3 = idx4.reshape(_NW, 4, _CROWS, 128)
    vals = vals4.reshape(_NW, 4, _CHUNK_G * 4)
    zeros = jnp.zeros((_ZWORDS,), jnp.float32)
    part = _sc_scatter(idx3, vals, zeros)
    rgb = _run_final(part.reshape(_NCORES, 4, _NPIX // 1024, 8, 128))
    return rgb.T.reshape(_S, _S, 3)
